# baseline (speedup 1.0000x reference)
import functools

import jax
import jax.numpy as jnp
from jax import lax
from jax.experimental import pallas as pl
from jax.experimental.pallas import tpu as pltpu
from jax.experimental.pallas import tpu_sc as plsc

N = 10000
NP = 10240
E = 320000
EP = 320000
TRASH = N
NC = 2
NS = 16
NW = NC * NS
EW = EP // NW
CH = 80
NCH = EW // CH
NG = 5
G = NCH // NG
EP1 = 322560
CH1 = 96
NG1 = 5
G1 = (EP1 // NW) // (CH1 * NG1)
RT = NP // NS
BLK = 1000
F32 = jnp.float32


def _mesh():
    return plsc.VectorSubcoreMesh(core_axis_name="c", subcore_axis_name="s")


def _deg_call(dst_r):
    @functools.partial(
        pl.kernel,
        out_type=jax.ShapeDtypeStruct((NC, NP), F32),
        mesh=_mesh(),
        scratch_types=[
            pltpu.VMEM((NG, G, CH), jnp.int32),
            pltpu.VMEM((CH,), F32),
            pltpu.VMEM((RT,), F32),
            pltpu.VMEM_SHARED((NP,), F32),
            pltpu.SemaphoreType.DMA,
        ],
    )
    def deg_k(dst_hbm, out_hbm, dst_v, ones_v, zb, acc, sem):
        c = lax.axis_index("c")
        s = lax.axis_index("s")
        wid = s * NC + c

        def zrow(i, carry):
            zb[pl.ds(i * 16, 16)] = jnp.zeros((16,), F32)
            return carry

        lax.fori_loop(0, RT // 16, zrow, 0)
        pltpu.sync_copy(zb, acc.at[pl.ds(s * RT, RT)])
        pltpu.sync_copy(dst_hbm.at[wid], dst_v)
        for i in range(CH // 16):
            ones_v[pl.ds(i * 16, 16)] = jnp.ones((16,), F32)
        plsc.subcore_barrier()

        for gi in range(NG):
            def body(j, carry):
                pltpu.sync_copy(ones_v, acc.at[dst_v.at[gi, j]], add=True)
                return carry

            lax.fori_loop(0, G, body, 0)
        plsc.subcore_barrier()
        pltpu.sync_copy(acc.at[pl.ds(s * RT, RT)], out_hbm.at[c, pl.ds(s * RT, RT)])

    return deg_k(dst_r)


def _scatter_call(y, src_r, dst_r, f, nbuf, ng, g, ch):
    @functools.partial(
        pl.kernel,
        out_type=jax.ShapeDtypeStruct((NC, NP, f), F32),
        mesh=_mesh(),
        scratch_types=(
            [pltpu.VMEM((2, g, ch), jnp.int32)] * 2
            + [pltpu.VMEM((ch, f), F32)] * nbuf
            + [pltpu.VMEM_SHARED((NP, f), F32)]
            + [pltpu.SemaphoreType.DMA] * (2 * nbuf + 2)
        ),
        compiler_params=pltpu.CompilerParams(use_tc_tiling_on_sc=(f == 128)),
    )
    def scat_k(y_hbm, src_hbm, dst_hbm, out_hbm, src_v, dst_v, *rest):
        bufs = list(rest[:nbuf])
        acc = rest[nbuf]
        gs = list(rest[nbuf + 1:2 * nbuf + 1])
        ss = list(rest[2 * nbuf + 1:3 * nbuf + 1])
        i0, i1 = rest[3 * nbuf + 1], rest[3 * nbuf + 2]
        c = lax.axis_index("c")
        s = lax.axis_index("s")
        wid = s * NC + c

        def zrow(r, carry):
            for k in range(f // 16):
                bufs[0][r, pl.ds(k * 16, 16)] = jnp.zeros((16,), F32)
            return carry

        lax.fori_loop(0, ch, zrow, 0)
        for t in range(RT // ch):
            pltpu.sync_copy(bufs[0], acc.at[pl.ds(s * RT + t * ch, ch)])
        rem = RT % ch
        if rem:
            pltpu.sync_copy(bufs[0].at[pl.ds(0, rem)],
                            acc.at[pl.ds(s * RT + (RT // ch) * ch, rem)])
        plsc.subcore_barrier()

        pltpu.sync_copy(src_hbm.at[wid, 0], src_v.at[0])
        pltpu.sync_copy(dst_hbm.at[wid, 0], dst_v.at[0])
        for gi in range(ng):
            p = gi % 2
            if gi + 1 < ng:
                pf0 = pltpu.async_copy(src_hbm.at[wid, gi + 1], src_v.at[1 - p], i0)
                pf1 = pltpu.async_copy(dst_hbm.at[wid, gi + 1], dst_v.at[1 - p], i1)

            def gather(j, r, sem):
                return pltpu.async_copy(y_hbm.at[src_v.at[p, j]], r, sem)

            def scat(j, r, sem):
                pltpu.async_copy(r, acc.at[dst_v.at[p, j]], sem, add=True)

            def drain(r, sem):
                pltpu.make_async_copy(r, acc.at[dst_v.at[p, 0]], sem).wait()

            hs = [gather(b, bufs[b], gs[b]) for b in range(nbuf)]
            for b in range(nbuf):
                hs[b].wait()
                scat(b, bufs[b], ss[b])

            def rot(i, carry):
                j = nbuf * i
                aa = []
                for b in range(nbuf):
                    drain(bufs[b], ss[b])
                    aa.append(gather(j + b, bufs[b], gs[b]))
                for b in range(nbuf):
                    aa[b].wait()
                    scat(j + b, bufs[b], ss[b])
                return carry

            lax.fori_loop(1, g // nbuf, rot, 0)
            for j in range(nbuf * (g // nbuf), g):
                drain(bufs[0], ss[0])
                hT = gather(j, bufs[0], gs[0])
                hT.wait()
                scat(j, bufs[0], ss[0])
            for b in range(nbuf):
                drain(bufs[b], ss[b])
            if gi + 1 < ng:
                pf0.wait()
                pf1.wait()
        plsc.subcore_barrier()
        pltpu.sync_copy(acc.at[pl.ds(s * RT, RT)], out_hbm.at[c, pl.ds(s * RT, RT)])

    return scat_k(y, src_r, dst_r)


def _tc0(x, w1):
    def body(xref, wref, yref):
        yref[...] = jnp.dot(xref[...], wref[...], preferred_element_type=F32)

    return pl.pallas_call(
        body,
        grid=(N // BLK,),
        in_specs=[
            pl.BlockSpec((BLK, 128), lambda i: (i, 0)),
            pl.BlockSpec((128, 128), lambda i: (0, 0)),
        ],
        out_specs=pl.BlockSpec((BLK, 128), lambda i: (i, 0)),
        out_shape=jax.ShapeDtypeStruct((N, 128), F32),
    )(x, w1)


def _tc1(deg_t, xw):
    def body(dref, xwref, yref, dinvref):
        d = dref[...]
        dinv = lax.rsqrt(1.0 + d[:, 0:1] + d[:, 1:2])
        yref[...] = dinv * xwref[...]
        dinvref[...] = dinv

    return pl.pallas_call(
        body,
        grid=(N // BLK,),
        in_specs=[
            pl.BlockSpec((BLK, 2), lambda i: (i, 0)),
            pl.BlockSpec((BLK, 128), lambda i: (i, 0)),
        ],
        out_specs=[
            pl.BlockSpec((BLK, 128), lambda i: (i, 0)),
            pl.BlockSpec((BLK, 1), lambda i: (i, 0)),
        ],
        out_shape=[
            jax.ShapeDtypeStruct((N, 128), F32),
            jax.ShapeDtypeStruct((N, 1), F32),
        ],
    )(deg_t, xw)


def _tc2(s1p, y1, dinv, b1, w2):
    def body(s0r, s1r, y1r, dr, br, wr, outr):
        dv = dr[...]
        h = jnp.maximum(dv * (s0r[0] + s1r[0] + y1r[...]) + br[...], 0.0)
        outr[...] = jnp.dot(dv * h, wr[...], preferred_element_type=F32)

    return pl.pallas_call(
        body,
        grid=(N // BLK,),
        in_specs=[
            pl.BlockSpec((1, BLK, 128), lambda i: (0, i, 0)),
            pl.BlockSpec((1, BLK, 128), lambda i: (1, i, 0)),
            pl.BlockSpec((BLK, 128), lambda i: (i, 0)),
            pl.BlockSpec((BLK, 1), lambda i: (i, 0)),
            pl.BlockSpec((1, 128), lambda i: (0, 0)),
            pl.BlockSpec((128, 64), lambda i: (0, 0)),
        ],
        out_specs=pl.BlockSpec((BLK, 64), lambda i: (i, 0)),
        out_shape=jax.ShapeDtypeStruct((N, 64), F32),
    )(s1p, s1p, y1, dinv, b1, w2)


def _tc3(s2p, y2, dinv, b2):
    def body(s0r, s1r, y2r, dr, br, outr):
        outr[...] = dr[...] * (s0r[0] + s1r[0] + y2r[...]) + br[...]

    return pl.pallas_call(
        body,
        grid=(N // BLK,),
        in_specs=[
            pl.BlockSpec((1, BLK, 64), lambda i: (0, i, 0)),
            pl.BlockSpec((1, BLK, 64), lambda i: (1, i, 0)),
            pl.BlockSpec((BLK, 64), lambda i: (i, 0)),
            pl.BlockSpec((BLK, 1), lambda i: (i, 0)),
            pl.BlockSpec((1, 64), lambda i: (0, 0)),
        ],
        out_specs=pl.BlockSpec((BLK, 64), lambda i: (i, 0)),
        out_shape=jax.ShapeDtypeStruct((N, 64), F32),
    )(s2p, s2p, y2, dinv, b2)


def kernel(x, edge_index, W1, b1, W2, b2):
    src_r = edge_index[0].reshape(NW, NG, G, CH)
    dst_r = edge_index[1].reshape(NW, NG, G, CH)
    pad = EP1 - E
    src1 = jnp.concatenate(
        [edge_index[0], jnp.zeros((pad,), jnp.int32)]).reshape(NW, NG1, G1, CH1)
    trash = N + jnp.arange(pad, dtype=jnp.int32) % (NP - N)
    dst1 = jnp.concatenate([edge_index[1], trash]).reshape(NW, NG1, G1, CH1)

    degp = _deg_call(dst_r)
    deg_t = jnp.transpose(degp)[:N]

    xw = _tc0(x, W1)
    y1, dinv = _tc1(deg_t, xw)
    s1p = _scatter_call(y1, src1, dst1, 128, 3, NG1, G1, CH1)
    y2 = _tc2(s1p, y1, dinv, b1.reshape(1, 128), W2)
    s2p = _scatter_call(y2, src_r, dst_r, 64, 5, NG, G, CH)
    return _tc3(s2p, y2, dinv, b2.reshape(1, 64))


# speedup vs baseline: 1.4074x; 1.4074x over previous
import functools

import jax
import jax.numpy as jnp
from jax import lax
from jax.experimental import pallas as pl
from jax.experimental.pallas import tpu as pltpu
from jax.experimental.pallas import tpu_sc as plsc

N = 10000
NP = 10240
E = 320000
EP = 320000
TRASH = N
NC = 2
NS = 16
NW = NC * NS
EW = EP // NW
CH = 80
NCH = EW // CH
NG = 5
G = NCH // NG
RT = NP // NS
BLK = 1000
F32 = jnp.float32


def _mesh():
    return plsc.VectorSubcoreMesh(core_axis_name="c", subcore_axis_name="s")


def _deg_call(dst_r):
    @functools.partial(
        pl.kernel,
        out_type=jax.ShapeDtypeStruct((NC, NP), F32),
        mesh=_mesh(),
        scratch_types=[
            pltpu.VMEM((NG, G, CH), jnp.int32),
            pltpu.VMEM((CH,), F32),
            pltpu.VMEM((RT,), F32),
            pltpu.VMEM_SHARED((NP,), F32),
            pltpu.SemaphoreType.DMA,
        ],
    )
    def deg_k(dst_hbm, out_hbm, dst_v, ones_v, zb, acc, sem):
        c = lax.axis_index("c")
        s = lax.axis_index("s")
        wid = s * NC + c

        def zrow(i, carry):
            zb[pl.ds(i * 16, 16)] = jnp.zeros((16,), F32)
            return carry

        lax.fori_loop(0, RT // 16, zrow, 0)
        pltpu.sync_copy(zb, acc.at[pl.ds(s * RT, RT)])
        pltpu.sync_copy(dst_hbm.at[wid], dst_v)
        for i in range(CH // 16):
            ones_v[pl.ds(i * 16, 16)] = jnp.ones((16,), F32)
        plsc.subcore_barrier()

        for gi in range(NG):
            def body(j, carry):
                pltpu.sync_copy(ones_v, acc.at[dst_v.at[gi, j]], add=True)
                return carry

            lax.fori_loop(0, G, body, 0)
        plsc.subcore_barrier()
        pltpu.sync_copy(acc.at[pl.ds(s * RT, RT)], out_hbm.at[c, pl.ds(s * RT, RT)])

    return deg_k(dst_r)


def _scatter_call(y, src_r, dst_r, f, nbuf, ng, g, ch):
    @functools.partial(
        pl.kernel,
        out_type=jax.ShapeDtypeStruct((NC, NP, f), F32),
        mesh=_mesh(),
        scratch_types=(
            [pltpu.VMEM((2, g, ch), jnp.int32)] * 2
            + [pltpu.VMEM((ch, f), F32)] * nbuf
            + [pltpu.VMEM_SHARED((NP, f), F32)]
            + [pltpu.SemaphoreType.DMA] * (2 * nbuf + 2)
        ),
        compiler_params=pltpu.CompilerParams(use_tc_tiling_on_sc=(f == 128)),
    )
    def scat_k(y_hbm, src_hbm, dst_hbm, out_hbm, src_v, dst_v, *rest):
        bufs = list(rest[:nbuf])
        acc = rest[nbuf]
        gs = list(rest[nbuf + 1:2 * nbuf + 1])
        ss = list(rest[2 * nbuf + 1:3 * nbuf + 1])
        i0, i1 = rest[3 * nbuf + 1], rest[3 * nbuf + 2]
        c = lax.axis_index("c")
        s = lax.axis_index("s")
        wid = s * NC + c

        def zrow(r, carry):
            for k in range(f // 16):
                bufs[0][r, pl.ds(k * 16, 16)] = jnp.zeros((16,), F32)
            return carry

        lax.fori_loop(0, ch, zrow, 0)
        for t in range(RT // ch):
            pltpu.sync_copy(bufs[0], acc.at[pl.ds(s * RT + t * ch, ch)])
        rem = RT % ch
        if rem:
            pltpu.sync_copy(bufs[0].at[pl.ds(0, rem)],
                            acc.at[pl.ds(s * RT + (RT // ch) * ch, rem)])
        plsc.subcore_barrier()

        pltpu.sync_copy(src_hbm.at[wid, 0], src_v.at[0])
        pltpu.sync_copy(dst_hbm.at[wid, 0], dst_v.at[0])
        for gi in range(ng):
            p = gi % 2
            if gi + 1 < ng:
                pf0 = pltpu.async_copy(src_hbm.at[wid, gi + 1], src_v.at[1 - p], i0)
                pf1 = pltpu.async_copy(dst_hbm.at[wid, gi + 1], dst_v.at[1 - p], i1)

            def gather(j, r, sem):
                return pltpu.async_copy(y_hbm.at[src_v.at[p, j]], r, sem)

            def scat(j, r, sem):
                pltpu.async_copy(r, acc.at[dst_v.at[p, j]], sem, add=True)

            def drain(r, sem):
                pltpu.make_async_copy(r, acc.at[dst_v.at[p, 0]], sem).wait()

            hs = [gather(b, bufs[b], gs[b]) for b in range(nbuf)]
            for b in range(nbuf):
                hs[b].wait()
                scat(b, bufs[b], ss[b])

            def rot(i, carry):
                j = nbuf * i
                aa = []
                for b in range(nbuf):
                    drain(bufs[b], ss[b])
                    aa.append(gather(j + b, bufs[b], gs[b]))
                for b in range(nbuf):
                    aa[b].wait()
                    scat(j + b, bufs[b], ss[b])
                return carry

            lax.fori_loop(1, g // nbuf, rot, 0)
            for j in range(nbuf * (g // nbuf), g):
                drain(bufs[0], ss[0])
                hT = gather(j, bufs[0], gs[0])
                hT.wait()
                scat(j, bufs[0], ss[0])
            for b in range(nbuf):
                drain(bufs[b], ss[b])
            if gi + 1 < ng:
                pf0.wait()
                pf1.wait()
        plsc.subcore_barrier()
        pltpu.sync_copy(acc.at[pl.ds(s * RT, RT)], out_hbm.at[c, pl.ds(s * RT, RT)])

    return scat_k(y, src_r, dst_r)


def _tc0(x, w1):
    def body(xref, wref, yref):
        yref[...] = jnp.dot(xref[...], wref[...], preferred_element_type=F32)

    return pl.pallas_call(
        body,
        grid=(N // BLK,),
        in_specs=[
            pl.BlockSpec((BLK, 128), lambda i: (i, 0)),
            pl.BlockSpec((128, 128), lambda i: (0, 0)),
        ],
        out_specs=pl.BlockSpec((BLK, 128), lambda i: (i, 0)),
        out_shape=jax.ShapeDtypeStruct((N, 128), F32),
    )(x, w1)


def _tc1(deg_t, xw):
    def body(dref, xwref, yref, dinvref):
        d = dref[...]
        dinv = lax.rsqrt(1.0 + d[:, 0:1] + d[:, 1:2])
        yref[...] = dinv * xwref[...]
        dinvref[...] = dinv

    return pl.pallas_call(
        body,
        grid=(N // BLK,),
        in_specs=[
            pl.BlockSpec((BLK, 2), lambda i: (i, 0)),
            pl.BlockSpec((BLK, 128), lambda i: (i, 0)),
        ],
        out_specs=[
            pl.BlockSpec((BLK, 128), lambda i: (i, 0)),
            pl.BlockSpec((BLK, 1), lambda i: (i, 0)),
        ],
        out_shape=[
            jax.ShapeDtypeStruct((N, 128), F32),
            jax.ShapeDtypeStruct((N, 1), F32),
        ],
    )(deg_t, xw)


def _tc2(s1p, y1, dinv, b1, w2):
    def body(s0r, s1r, y1r, dr, br, wr, outr):
        dv = dr[...]
        h = jnp.maximum(dv * (s0r[0] + s1r[0] + y1r[...]) + br[...], 0.0)
        outr[...] = jnp.dot(dv * h, wr[...], preferred_element_type=F32)

    return pl.pallas_call(
        body,
        grid=(N // BLK,),
        in_specs=[
            pl.BlockSpec((1, BLK, 128), lambda i: (0, i, 0)),
            pl.BlockSpec((1, BLK, 128), lambda i: (1, i, 0)),
            pl.BlockSpec((BLK, 128), lambda i: (i, 0)),
            pl.BlockSpec((BLK, 1), lambda i: (i, 0)),
            pl.BlockSpec((1, 128), lambda i: (0, 0)),
            pl.BlockSpec((128, 64), lambda i: (0, 0)),
        ],
        out_specs=pl.BlockSpec((BLK, 64), lambda i: (i, 0)),
        out_shape=jax.ShapeDtypeStruct((N, 64), F32),
    )(s1p, s1p, y1, dinv, b1, w2)


def _tc3(s2p, y2, dinv, b2):
    def body(s0r, s1r, y2r, dr, br, outr):
        outr[...] = dr[...] * (s0r[0] + s1r[0] + y2r[...]) + br[...]

    return pl.pallas_call(
        body,
        grid=(N // BLK,),
        in_specs=[
            pl.BlockSpec((1, BLK, 64), lambda i: (0, i, 0)),
            pl.BlockSpec((1, BLK, 64), lambda i: (1, i, 0)),
            pl.BlockSpec((BLK, 64), lambda i: (i, 0)),
            pl.BlockSpec((BLK, 1), lambda i: (i, 0)),
            pl.BlockSpec((1, 64), lambda i: (0, 0)),
        ],
        out_specs=pl.BlockSpec((BLK, 64), lambda i: (i, 0)),
        out_shape=jax.ShapeDtypeStruct((N, 64), F32),
    )(s2p, s2p, y2, dinv, b2)


def kernel(x, edge_index, W1, b1, W2, b2):
    src_r = edge_index[0].reshape(NW, NG, G, CH)
    dst_r = edge_index[1].reshape(NW, NG, G, CH)
    degp = _deg_call(dst_r)
    deg_t = jnp.transpose(degp)[:N]

    xw = _tc0(x, W1)
    y1, dinv = _tc1(deg_t, xw)
    s1p = _scatter_call(y1, src_r, dst_r, 128, 3, NG, G, CH)
    y2 = _tc2(s1p, y1, dinv, b1.reshape(1, 128), W2)
    s2p = _scatter_call(y2, src_r, dst_r, 64, 5, NG, G, CH)
    return _tc3(s2p, y2, dinv, b2.reshape(1, 64))


# speedup vs baseline: 1.4177x; 1.0073x over previous
import functools

import jax
import jax.numpy as jnp
from jax import lax
from jax.experimental import pallas as pl
from jax.experimental.pallas import tpu as pltpu
from jax.experimental.pallas import tpu_sc as plsc

N = 10000
NP = 10240
E = 320000
EP = 320000
TRASH = N
NC = 2
NS = 16
NW = NC * NS
EW = EP // NW
CH = 80
NCH = EW // CH
NG = 5
G = NCH // NG
RT = NP // NS
BLK = 1000
F32 = jnp.float32


def _mesh():
    return plsc.VectorSubcoreMesh(core_axis_name="c", subcore_axis_name="s")


def _deg_call(dst_r):
    @functools.partial(
        pl.kernel,
        out_type=jax.ShapeDtypeStruct((NC, NP), F32),
        mesh=_mesh(),
        scratch_types=[
            pltpu.VMEM((NG, G, CH), jnp.int32),
            pltpu.VMEM((CH,), F32),
            pltpu.VMEM((RT,), F32),
            pltpu.VMEM_SHARED((NP,), F32),
            pltpu.SemaphoreType.DMA,
        ],
    )
    def deg_k(dst_hbm, out_hbm, dst_v, ones_v, zb, acc, sem):
        c = lax.axis_index("c")
        s = lax.axis_index("s")
        wid = s * NC + c

        def zrow(i, carry):
            zb[pl.ds(i * 16, 16)] = jnp.zeros((16,), F32)
            return carry

        lax.fori_loop(0, RT // 16, zrow, 0)
        pltpu.sync_copy(zb, acc.at[pl.ds(s * RT, RT)])
        pltpu.sync_copy(dst_hbm.at[wid], dst_v)
        for i in range(CH // 16):
            ones_v[pl.ds(i * 16, 16)] = jnp.ones((16,), F32)
        plsc.subcore_barrier()

        for gi in range(NG):
            def body(j, carry):
                pltpu.sync_copy(ones_v, acc.at[dst_v.at[gi, j]], add=True)
                return carry

            lax.fori_loop(0, G, body, 0)
        plsc.subcore_barrier()
        pltpu.sync_copy(acc.at[pl.ds(s * RT, RT)], out_hbm.at[c, pl.ds(s * RT, RT)])

    return deg_k(dst_r)


def _scatter_call(y, src_r, dst_r, f, nbuf, ng, g, ch):
    @functools.partial(
        pl.kernel,
        out_type=jax.ShapeDtypeStruct((NC, NP, f), F32),
        mesh=_mesh(),
        scratch_types=(
            [pltpu.VMEM((2, g, ch), jnp.int32)] * 2
            + [pltpu.VMEM((ch, f), F32)] * nbuf
            + [pltpu.VMEM_SHARED((NP, f), F32)]
            + [pltpu.SemaphoreType.DMA] * (2 * nbuf + 2)
        ),
        compiler_params=pltpu.CompilerParams(use_tc_tiling_on_sc=(f == 128)),
    )
    def scat_k(y_hbm, src_hbm, dst_hbm, out_hbm, src_v, dst_v, *rest):
        bufs = list(rest[:nbuf])
        acc = rest[nbuf]
        gs = list(rest[nbuf + 1:2 * nbuf + 1])
        ss = list(rest[2 * nbuf + 1:3 * nbuf + 1])
        i0, i1 = rest[3 * nbuf + 1], rest[3 * nbuf + 2]
        c = lax.axis_index("c")
        s = lax.axis_index("s")
        wid = s * NC + c

        def zrow(r, carry):
            for k in range(f // 16):
                bufs[0][r, pl.ds(k * 16, 16)] = jnp.zeros((16,), F32)
            return carry

        lax.fori_loop(0, ch, zrow, 0)
        for t in range(RT // ch):
            pltpu.sync_copy(bufs[0], acc.at[pl.ds(s * RT + t * ch, ch)])
        rem = RT % ch
        if rem:
            pltpu.sync_copy(bufs[0].at[pl.ds(0, rem)],
                            acc.at[pl.ds(s * RT + (RT // ch) * ch, rem)])
        plsc.subcore_barrier()

        pltpu.sync_copy(src_hbm.at[wid, 0], src_v.at[0])
        pltpu.sync_copy(dst_hbm.at[wid, 0], dst_v.at[0])
        for gi in range(ng):
            p = gi % 2
            if gi + 1 < ng:
                pf0 = pltpu.async_copy(src_hbm.at[wid, gi + 1], src_v.at[1 - p], i0)
                pf1 = pltpu.async_copy(dst_hbm.at[wid, gi + 1], dst_v.at[1 - p], i1)

            def gather(j, r, sem):
                return pltpu.async_copy(y_hbm.at[src_v.at[p, j]], r, sem)

            def scat(j, r, sem):
                pltpu.async_copy(r, acc.at[dst_v.at[p, j]], sem, add=True)

            def drain(r, sem):
                pltpu.make_async_copy(r, acc.at[dst_v.at[p, 0]], sem).wait()

            hs = []
            for b in range(nbuf):
                if gi > 0:
                    drain(bufs[b], ss[b])
                hs.append(gather(b, bufs[b], gs[b]))
            for b in range(nbuf):
                hs[b].wait()
                scat(b, bufs[b], ss[b])

            def rot(i, carry):
                j = nbuf * i
                aa = []
                for b in range(nbuf):
                    drain(bufs[b], ss[b])
                    aa.append(gather(j + b, bufs[b], gs[b]))
                for b in range(nbuf):
                    aa[b].wait()
                    scat(j + b, bufs[b], ss[b])
                return carry

            if g > 2 * nbuf:
                lax.fori_loop(1, g // nbuf, rot, 0)
            else:
                for i in range(1, g // nbuf):
                    rot(i, 0)
            for j in range(nbuf * (g // nbuf), g):
                drain(bufs[0], ss[0])
                hT = gather(j, bufs[0], gs[0])
                hT.wait()
                scat(j, bufs[0], ss[0])
            if gi + 1 < ng:
                pf0.wait()
                pf1.wait()
        for b in range(nbuf):
            drain(bufs[b], ss[b])
        plsc.subcore_barrier()
        pltpu.sync_copy(acc.at[pl.ds(s * RT, RT)], out_hbm.at[c, pl.ds(s * RT, RT)])

    return scat_k(y, src_r, dst_r)


def _tc0(x, w1):
    def body(xref, wref, yref):
        yref[...] = jnp.dot(xref[...], wref[...], preferred_element_type=F32)

    return pl.pallas_call(
        body,
        grid=(N // BLK,),
        in_specs=[
            pl.BlockSpec((BLK, 128), lambda i: (i, 0)),
            pl.BlockSpec((128, 128), lambda i: (0, 0)),
        ],
        out_specs=pl.BlockSpec((BLK, 128), lambda i: (i, 0)),
        out_shape=jax.ShapeDtypeStruct((N, 128), F32),
    )(x, w1)


def _tc1(deg_t, xw):
    def body(dref, xwref, yref, dinvref):
        d = dref[...]
        dinv = lax.rsqrt(1.0 + d[:, 0:1] + d[:, 1:2])
        yref[...] = dinv * xwref[...]
        dinvref[...] = dinv

    return pl.pallas_call(
        body,
        grid=(N // BLK,),
        in_specs=[
            pl.BlockSpec((BLK, 2), lambda i: (i, 0)),
            pl.BlockSpec((BLK, 128), lambda i: (i, 0)),
        ],
        out_specs=[
            pl.BlockSpec((BLK, 128), lambda i: (i, 0)),
            pl.BlockSpec((BLK, 1), lambda i: (i, 0)),
        ],
        out_shape=[
            jax.ShapeDtypeStruct((N, 128), F32),
            jax.ShapeDtypeStruct((N, 1), F32),
        ],
    )(deg_t, xw)


def _tc2(s1p, y1, dinv, b1, w2):
    def body(s0r, s1r, y1r, dr, br, wr, outr):
        dv = dr[...]
        h = jnp.maximum(dv * (s0r[0] + s1r[0] + y1r[...]) + br[...], 0.0)
        outr[...] = jnp.dot(dv * h, wr[...], preferred_element_type=F32)

    return pl.pallas_call(
        body,
        grid=(N // BLK,),
        in_specs=[
            pl.BlockSpec((1, BLK, 128), lambda i: (0, i, 0)),
            pl.BlockSpec((1, BLK, 128), lambda i: (1, i, 0)),
            pl.BlockSpec((BLK, 128), lambda i: (i, 0)),
            pl.BlockSpec((BLK, 1), lambda i: (i, 0)),
            pl.BlockSpec((1, 128), lambda i: (0, 0)),
            pl.BlockSpec((128, 64), lambda i: (0, 0)),
        ],
        out_specs=pl.BlockSpec((BLK, 64), lambda i: (i, 0)),
        out_shape=jax.ShapeDtypeStruct((N, 64), F32),
    )(s1p, s1p, y1, dinv, b1, w2)


def _tc3(s2p, y2, dinv, b2):
    def body(s0r, s1r, y2r, dr, br, outr):
        outr[...] = dr[...] * (s0r[0] + s1r[0] + y2r[...]) + br[...]

    return pl.pallas_call(
        body,
        grid=(N // BLK,),
        in_specs=[
            pl.BlockSpec((1, BLK, 64), lambda i: (0, i, 0)),
            pl.BlockSpec((1, BLK, 64), lambda i: (1, i, 0)),
            pl.BlockSpec((BLK, 64), lambda i: (i, 0)),
            pl.BlockSpec((BLK, 1), lambda i: (i, 0)),
            pl.BlockSpec((1, 64), lambda i: (0, 0)),
        ],
        out_specs=pl.BlockSpec((BLK, 64), lambda i: (i, 0)),
        out_shape=jax.ShapeDtypeStruct((N, 64), F32),
    )(s2p, s2p, y2, dinv, b2)


def kernel(x, edge_index, W1, b1, W2, b2):
    src_r = edge_index[0].reshape(NW, NG, G, CH)
    dst_r = edge_index[1].reshape(NW, NG, G, CH)
    src_g = edge_index[0].reshape(NW, G, NG, CH)
    dst_g = edge_index[1].reshape(NW, G, NG, CH)
    degp = _deg_call(dst_r)
    deg_t = jnp.transpose(degp)[:N]

    xw = _tc0(x, W1)
    y1, dinv = _tc1(deg_t, xw)
    s1p = _scatter_call(y1, src_r, dst_r, 128, 3, NG, G, CH)
    y2 = _tc2(s1p, y1, dinv, b1.reshape(1, 128), W2)
    s2p = _scatter_call(y2, src_g, dst_g, 64, 5, G, NG, CH)
    return _tc3(s2p, y2, dinv, b2.reshape(1, 64))
